# baseline (speedup 1.0000x reference)
import functools

import jax
import jax.numpy as jnp
from jax import lax
from jax.experimental import pallas as pl
from jax.experimental.pallas import tpu as pltpu
from jax.experimental.pallas import tpu_sc as plsc

_INFO = plsc.get_sparse_core_info()
_NC = _INFO.num_cores
_NS = _INFO.num_subcores
_NW = _NC * _NS


@functools.lru_cache(maxsize=None)
def _make_gather(num_rows: int, d_model: int):
    assert num_rows % _NW == 0
    b_per_w = num_rows // _NW
    chunk = 32
    assert b_per_w % chunk == 0
    n_chunks = b_per_w // chunk

    mesh = plsc.VectorSubcoreMesh(core_axis_name="c", subcore_axis_name="s")

    @functools.partial(
        pl.kernel,
        mesh=mesh,
        out_type=jax.ShapeDtypeStruct((num_rows, d_model), jnp.float32),
        scratch_types=[
            pltpu.VMEM((n_chunks, chunk), jnp.int32),
            pltpu.VMEM((chunk, d_model), jnp.float32),
            pltpu.VMEM((chunk, d_model), jnp.float32),
            pltpu.SemaphoreType.DMA,
            pltpu.SemaphoreType.DMA,
            pltpu.SemaphoreType.DMA,
            pltpu.SemaphoreType.DMA,
        ],
    )
    def gather_kernel(idx_hbm, table_hbm, out_hbm,
                      idx_v, buf0, buf1, g0, g1, s0, s1):
        wid = lax.axis_index("s") * _NC + lax.axis_index("c")
        base = wid * b_per_w
        pltpu.sync_copy(idx_hbm.at[wid], idx_v)

        bufs = (buf0, buf1)
        gsems = (g0, g1)
        ssems = (s0, s1)

        def start_gather(c):
            return pltpu.async_copy(
                table_hbm.at[idx_v.at[c]],
                bufs[c % 2],
                gsems[c % 2],
            )

        def start_store(c):
            return pltpu.async_copy(
                bufs[c % 2],
                out_hbm.at[pl.ds(base + c * chunk, chunk)],
                ssems[c % 2],
            )

        gathers = [None] * n_chunks
        stores = [None] * n_chunks
        gathers[0] = start_gather(0)
        for c in range(n_chunks):
            if c + 1 < n_chunks:
                if c >= 1:
                    stores[c - 1].wait()
                gathers[c + 1] = start_gather(c + 1)
            gathers[c].wait()
            stores[c] = start_store(c)
        if n_chunks >= 2:
            stores[n_chunks - 2].wait()
        stores[n_chunks - 1].wait()

    return gather_kernel


def kernel(input_ids, table):
    batch, seq = input_ids.shape
    vocab, d_model = table.shape
    num_rows = batch * seq
    b_per_w = num_rows // _NW
    idx = input_ids.reshape(_NW, b_per_w // 32, 32).astype(jnp.int32)
    rows = _make_gather(num_rows, d_model)(idx, table)
    return rows.reshape(batch, seq, d_model)


# speedup vs baseline: 1.0220x; 1.0220x over previous
import functools

import jax
import jax.numpy as jnp
from jax import lax
from jax.experimental import pallas as pl
from jax.experimental.pallas import tpu as pltpu
from jax.experimental.pallas import tpu_sc as plsc

_INFO = plsc.get_sparse_core_info()
_NC = _INFO.num_cores
_NS = _INFO.num_subcores
_NW = _NC * _NS


@functools.lru_cache(maxsize=None)
def _make_gather(batch: int, seq: int, d_model: int):
    num_rows = batch * seq
    assert num_rows % _NW == 0
    b_per_w = num_rows // _NW
    assert seq % b_per_w == 0
    w_per_row = seq // b_per_w
    chunk = 32
    nbuf = 3
    assert b_per_w % chunk == 0
    n_chunks = b_per_w // chunk

    mesh = plsc.VectorSubcoreMesh(core_axis_name="c", subcore_axis_name="s")

    @functools.partial(
        pl.kernel,
        mesh=mesh,
        out_type=jax.ShapeDtypeStruct((num_rows, d_model), jnp.float32),
        scratch_types=[
            pltpu.VMEM((b_per_w,), jnp.int32),
            pltpu.VMEM((chunk, d_model), jnp.float32),
            pltpu.VMEM((chunk, d_model), jnp.float32),
            pltpu.VMEM((chunk, d_model), jnp.float32),
            pltpu.SemaphoreType.DMA,
            pltpu.SemaphoreType.DMA,
            pltpu.SemaphoreType.DMA,
            pltpu.SemaphoreType.DMA,
            pltpu.SemaphoreType.DMA,
            pltpu.SemaphoreType.DMA,
        ],
    )
    def gather_kernel(idx_hbm, table_hbm, out_hbm,
                      idx_v, buf0, buf1, buf2, g0, g1, g2, s0, s1, s2):
        wid = lax.axis_index("s") * _NC + lax.axis_index("c")
        base = wid * b_per_w
        pltpu.sync_copy(
            idx_hbm.at[wid // w_per_row,
                       pl.ds((wid % w_per_row) * b_per_w, b_per_w)],
            idx_v,
        )

        bufs = (buf0, buf1, buf2)
        gsems = (g0, g1, g2)
        ssems = (s0, s1, s2)

        def start_gather(c):
            return pltpu.async_copy(
                table_hbm.at[idx_v.at[pl.ds(c * chunk, chunk)]],
                bufs[c % nbuf],
                gsems[c % nbuf],
            )

        def start_store(c):
            return pltpu.async_copy(
                bufs[c % nbuf],
                out_hbm.at[pl.ds(base + c * chunk, chunk)],
                ssems[c % nbuf],
            )

        gathers = [None] * n_chunks
        stores = [None] * n_chunks
        for c in range(min(nbuf - 1, n_chunks)):
            gathers[c] = start_gather(c)
        for c in range(n_chunks):
            if c + nbuf - 1 < n_chunks:
                if c >= 1:
                    stores[c - 1].wait()
                gathers[c + nbuf - 1] = start_gather(c + nbuf - 1)
            gathers[c].wait()
            stores[c] = start_store(c)
        for c in range(max(0, n_chunks - nbuf), n_chunks):
            stores[c].wait()

    return gather_kernel


def kernel(input_ids, table):
    batch, seq = input_ids.shape
    vocab, d_model = table.shape
    rows = _make_gather(batch, seq, d_model)(
        input_ids.astype(jnp.int32), table)
    return rows.reshape(batch, seq, d_model)
